# baseline (speedup 1.0000x reference)
import functools

import jax
import jax.numpy as jnp
from jax.experimental import pallas as pl
from jax.experimental.pallas import tpu as pltpu


def _mm_gelu_kernel(x_ref, w_ref, o_ref):
    x = x_ref[...].astype(jnp.bfloat16)
    w = w_ref[...].astype(jnp.bfloat16)
    a = jnp.dot(x, w, preferred_element_type=jnp.float32).astype(jnp.bfloat16)
    c1 = jnp.bfloat16(0.7978845608028654)
    c2 = jnp.bfloat16(0.035677408136300125)
    half = jnp.bfloat16(0.5)
    z = a * (c1 + c2 * (a * a))
    ah = half * a
    o_ref[...] = (ah + ah * jnp.tanh(z)).astype(jnp.float32)


@functools.partial(jax.jit, static_argnames=("block_m", "block_n"))
def _run(flat, W, block_m, block_n):
    m, d = flat.shape
    grid = (m // block_m, d // block_n)
    return pl.pallas_call(
        _mm_gelu_kernel,
        grid=grid,
        in_specs=[
            pl.BlockSpec((block_m, d), lambda i, j: (i, 0)),
            pl.BlockSpec((d, block_n), lambda i, j: (0, j)),
        ],
        out_specs=pl.BlockSpec((block_m, block_n), lambda i, j: (i, j)),
        out_shape=jax.ShapeDtypeStruct((m, d), flat.dtype),
        compiler_params=pltpu.CompilerParams(
            dimension_semantics=("parallel", "arbitrary"),
        ),
    )(flat, W)


def kernel(flat, cu_seqlens, W):
    del cu_seqlens
    return _run(flat, W, 4096, 256)


# speedup vs baseline: 1.2604x; 1.2604x over previous
import functools

import jax
import jax.numpy as jnp
from jax.experimental import pallas as pl
from jax.experimental.pallas import tpu as pltpu


def _mm_gelu_kernel(x_ref, w_ref, o_ref):
    a = jnp.dot(x_ref[...], w_ref[...],
                preferred_element_type=jnp.float32).astype(jnp.bfloat16)
    c1 = jnp.bfloat16(0.7978845608028654)
    c2 = jnp.bfloat16(0.035677408136300125)
    half = jnp.bfloat16(0.5)
    z = a * (c1 + c2 * (a * a))
    ah = half * a
    o_ref[...] = (ah + ah * jnp.tanh(z)).astype(jnp.float32)


@functools.partial(jax.jit, static_argnames=("block_m",))
def _run(flat, W, block_m):
    m, d = flat.shape
    grid = (m // block_m,)
    return pl.pallas_call(
        _mm_gelu_kernel,
        grid=grid,
        in_specs=[
            pl.BlockSpec((block_m, d), lambda i: (i, 0)),
            pl.BlockSpec((d, d), lambda i: (0, 0)),
        ],
        out_specs=pl.BlockSpec((block_m, d), lambda i: (i, 0)),
        out_shape=jax.ShapeDtypeStruct((m, d), flat.dtype),
        compiler_params=pltpu.CompilerParams(
            dimension_semantics=("parallel",),
        ),
    )(flat, W)


def kernel(flat, cu_seqlens, W):
    del cu_seqlens
    return _run(flat, W, 4096)
